# baseline (speedup 1.0000x reference)
import functools

import jax
import jax.numpy as jnp
from jax import lax
from jax.experimental import pallas as pl
from jax.experimental.pallas import tpu as pltpu
from jax.experimental.pallas import tpu_sc as plsc

NUM_ROWS = 1000000
DIM = 64
BATCH = 4096
SEQ = 200
LANES = 128
NBUF = 2

_INFO = plsc.get_sparse_core_info()
NC = _INFO.num_cores
NS = _INFO.num_subcores
NW = NC * NS
NGRP = SEQ // NBUF


@functools.partial(
    pl.kernel,
    mesh=plsc.VectorSubcoreMesh(core_axis_name="c", subcore_axis_name="s"),
    compiler_params=pltpu.CompilerParams(
        use_tc_tiling_on_sc=True, needs_layout_passes=False
    ),
    out_type=jax.ShapeDtypeStruct((SEQ, DIM, BATCH), jnp.float32),
    scratch_types=[
        pltpu.VMEM((SEQ, LANES), jnp.int32),
    ]
    + [pltpu.VMEM((LANES,), jnp.int32) for _ in range(NBUF)]
    + [pltpu.VMEM((LANES, LANES), jnp.float32) for _ in range(NBUF)]
    + [pltpu.VMEM((DIM, LANES), jnp.float32) for _ in range(NBUF)]
    + [
        pltpu.SemaphoreType.DMA,
        pltpu.SemaphoreType.DMA,
    ],
)
def _emb_lookup(idx_hbm, table_hbm, out_hbm, idx_v, *rest):
    pidx = rest[0:NBUF]
    rows = rest[NBUF:2 * NBUF]
    obuf = rest[2 * NBUF:3 * NBUF]
    gsem, ssem = rest[3 * NBUF], rest[3 * NBUF + 1]

    wid = lax.axis_index("s") * NC + lax.axis_index("c")
    obase = wid * LANES

    pltpu.sync_copy(idx_hbm.at[wid], idx_v)

    lane16 = lax.iota(jnp.int32, 16)

    def compute_pidx(k, t):
        for lg in range(LANES // 16):
            v = idx_v[t, pl.ds(lg * 16, 16)]
            pidx[k][pl.ds(lg * 16, 16)] = v >> 1

    def transpose_extract(k, t):
        for lg in range(LANES // 16):
            v = idx_v[t, pl.ds(lg * 16, 16)]
            half16 = (v & 1) << 6
            row_idx = lane16 + (lg * 16)
            for d in range(DIM):
                obuf[k][d, pl.ds(lg * 16, 16)] = plsc.load_gather(
                    rows[k], [row_idx, half16 + d]
                )

    def body(g, _):
        t0 = g * NBUF
        handles = []
        for k in range(NBUF):
            compute_pidx(k, t0 + k)
            handles.append(
                pltpu.async_copy(table_hbm.at[pidx[k]], rows[k], gsem)
            )
        for k in range(NBUF):
            @pl.when(g > 0)
            def _():
                pltpu.make_async_copy(
                    obuf[k],
                    out_hbm.at[t0 + k - NBUF, :, pl.ds(obase, LANES)],
                    ssem,
                ).wait()

            handles[k].wait()
            transpose_extract(k, t0 + k)
            pltpu.async_copy(
                obuf[k], out_hbm.at[t0 + k, :, pl.ds(obase, LANES)], ssem
            )
        return 0

    lax.fori_loop(0, NGRP, body, 0)
    for k in range(NBUF):
        pltpu.make_async_copy(
            obuf[k], out_hbm.at[SEQ - NBUF + k, :, pl.ds(obase, LANES)], ssem
        ).wait()


def kernel(token_ids, weight):
    idx = (
        token_ids.astype(jnp.int32)
        .T.reshape(SEQ, NW, LANES)
        .transpose(1, 0, 2)
    )
    table2 = weight.reshape(NUM_ROWS // 2, 2 * DIM)
    out = _emb_lookup(idx, table2)
    return out.transpose(2, 0, 1)


# speedup vs baseline: 2.3000x; 2.3000x over previous
import functools

import jax
import jax.numpy as jnp
from jax import lax
from jax.experimental import pallas as pl
from jax.experimental.pallas import tpu as pltpu
from jax.experimental.pallas import tpu_sc as plsc

NUM_ROWS = 1000000
DIM = 64
BATCH = 4096
SEQ = 200
LANES = 128
NBUF = 2

_INFO = plsc.get_sparse_core_info()
NC = _INFO.num_cores
NS = _INFO.num_subcores
NW = NC * NS
NGRP = SEQ // NBUF


@functools.partial(
    pl.kernel,
    mesh=plsc.VectorSubcoreMesh(core_axis_name="c", subcore_axis_name="s"),
    compiler_params=pltpu.CompilerParams(
        use_tc_tiling_on_sc=True, needs_layout_passes=False
    ),
    out_type=jax.ShapeDtypeStruct((SEQ, DIM, BATCH), jnp.float32),
    scratch_types=[
        pltpu.VMEM((SEQ, LANES), jnp.int32),
    ]
    + [pltpu.VMEM((LANES,), jnp.int32) for _ in range(NBUF)]
    + [pltpu.VMEM((LANES, LANES), jnp.float32) for _ in range(NBUF)]
    + [pltpu.VMEM((DIM, LANES), jnp.float32) for _ in range(NBUF)]
    + [
        pltpu.SemaphoreType.DMA,
        pltpu.SemaphoreType.DMA,
    ],
)
def _emb_lookup(idx_hbm, table_hbm, out_hbm, idx_v, *rest):
    pidx = rest[0:NBUF]
    rows = rest[NBUF:2 * NBUF]
    obuf = rest[2 * NBUF:3 * NBUF]
    gsem, ssem = rest[3 * NBUF], rest[3 * NBUF + 1]

    wid = lax.axis_index("s") * NC + lax.axis_index("c")
    obase = wid * LANES

    pltpu.sync_copy(idx_hbm.at[wid], idx_v)

    lane16 = lax.iota(jnp.int32, 16)

    def compute_pidx(k, t):
        for lg in range(LANES // 16):
            v = idx_v[t, pl.ds(lg * 16, 16)]
            pidx[k][pl.ds(lg * 16, 16)] = v >> 1

    def transpose_extract(k, t):
        for lg in range(LANES // 16):
            v = idx_v[t, pl.ds(lg * 16, 16)]
            half16 = (v & 1) << 6
            row_idx = lane16 + (lg * 16)

            @plsc.parallel_loop(0, DIM, unroll=8)
            def _(c):
                d_vec = (lane16 + c) & (DIM - 1)
                vals = plsc.load_gather(rows[k], [row_idx, half16 + d_vec])
                plsc.store_scatter(obuf[k], [d_vec, row_idx], vals)

    def body(g, _):
        t0 = g * NBUF
        handles = []
        for k in range(NBUF):
            compute_pidx(k, t0 + k)
            handles.append(
                pltpu.async_copy(table_hbm.at[pidx[k]], rows[k], gsem)
            )
        for k in range(NBUF):
            @pl.when(g > 0)
            def _():
                pltpu.make_async_copy(
                    obuf[k],
                    out_hbm.at[t0 + k - NBUF, :, pl.ds(obase, LANES)],
                    ssem,
                ).wait()

            handles[k].wait()
            transpose_extract(k, t0 + k)
            pltpu.async_copy(
                obuf[k], out_hbm.at[t0 + k, :, pl.ds(obase, LANES)], ssem
            )
        return 0

    lax.fori_loop(0, NGRP, body, 0)
    for k in range(NBUF):
        pltpu.make_async_copy(
            obuf[k], out_hbm.at[SEQ - NBUF + k, :, pl.ds(obase, LANES)], ssem
        ).wait()


def kernel(token_ids, weight):
    idx = (
        token_ids.astype(jnp.int32)
        .T.reshape(SEQ, NW, LANES)
        .transpose(1, 0, 2)
    )
    table2 = weight.reshape(NUM_ROWS // 2, 2 * DIM)
    out = _emb_lookup(idx, table2)
    return out.transpose(2, 0, 1)


# speedup vs baseline: 2.4554x; 1.0676x over previous
import functools

import jax
import jax.numpy as jnp
from jax import lax
from jax.experimental import pallas as pl
from jax.experimental.pallas import tpu as pltpu
from jax.experimental.pallas import tpu_sc as plsc

NUM_ROWS = 1000000
DIM = 64
BATCH = 4096
SEQ = 200
LANES = 128
NBUF = 4

_INFO = plsc.get_sparse_core_info()
NC = _INFO.num_cores
NS = _INFO.num_subcores
NW = NC * NS
NGRP = SEQ // NBUF


@functools.partial(
    pl.kernel,
    mesh=plsc.VectorSubcoreMesh(core_axis_name="c", subcore_axis_name="s"),
    compiler_params=pltpu.CompilerParams(
        use_tc_tiling_on_sc=True, needs_layout_passes=False
    ),
    out_type=jax.ShapeDtypeStruct((SEQ, DIM, BATCH), jnp.float32),
    scratch_types=[
        pltpu.VMEM((SEQ, LANES), jnp.int32),
    ]
    + [pltpu.VMEM((LANES,), jnp.int32) for _ in range(NBUF)]
    + [pltpu.VMEM((LANES, LANES), jnp.float32) for _ in range(NBUF)]
    + [pltpu.VMEM((DIM, LANES), jnp.float32) for _ in range(NBUF)]
    + [
        pltpu.SemaphoreType.DMA,
        pltpu.SemaphoreType.DMA,
    ],
)
def _emb_lookup(idx_hbm, table_hbm, out_hbm, idx_v, *rest):
    pidx = rest[0:NBUF]
    rows = rest[NBUF:2 * NBUF]
    obuf = rest[2 * NBUF:3 * NBUF]
    gsem, ssem = rest[3 * NBUF], rest[3 * NBUF + 1]

    wid = lax.axis_index("s") * NC + lax.axis_index("c")
    obase = wid * LANES

    pltpu.sync_copy(idx_hbm.at[wid], idx_v)

    lane16 = lax.iota(jnp.int32, 16)

    def compute_pidx(k, t):
        for lg in range(LANES // 16):
            v = idx_v[t, pl.ds(lg * 16, 16)]
            pidx[k][pl.ds(lg * 16, 16)] = v >> 1

    def transpose_extract(k, t):
        for lg in range(LANES // 16):
            v = idx_v[t, pl.ds(lg * 16, 16)]
            half16 = (v & 1) << 6
            row_idx = lane16 + (lg * 16)

            @plsc.parallel_loop(0, DIM, unroll=8)
            def _(c):
                d_vec = (lane16 + c) & (DIM - 1)
                vals = plsc.load_gather(rows[k], [row_idx, half16 + d_vec])
                plsc.store_scatter(obuf[k], [d_vec, row_idx], vals)

    def body(g, _):
        t0 = g * NBUF
        handles = []
        for k in range(NBUF):
            compute_pidx(k, t0 + k)
            handles.append(
                pltpu.async_copy(table_hbm.at[pidx[k]], rows[k], gsem)
            )
        for k in range(NBUF):
            @pl.when(g > 0)
            def _():
                pltpu.make_async_copy(
                    obuf[k],
                    out_hbm.at[t0 + k - NBUF, :, pl.ds(obase, LANES)],
                    ssem,
                ).wait()

            handles[k].wait()
            transpose_extract(k, t0 + k)
            pltpu.async_copy(
                obuf[k], out_hbm.at[t0 + k, :, pl.ds(obase, LANES)], ssem
            )
        return 0

    lax.fori_loop(0, NGRP, body, 0)
    for k in range(NBUF):
        pltpu.make_async_copy(
            obuf[k], out_hbm.at[SEQ - NBUF + k, :, pl.ds(obase, LANES)], ssem
        ).wait()


def kernel(token_ids, weight):
    idx = (
        token_ids.astype(jnp.int32)
        .T.reshape(SEQ, NW, LANES)
        .transpose(1, 0, 2)
    )
    table2 = weight.reshape(NUM_ROWS // 2, 2 * DIM)
    out = _emb_lookup(idx, table2)
    return out.transpose(2, 0, 1)


# speedup vs baseline: 2.4571x; 1.0007x over previous
import functools

import jax
import jax.numpy as jnp
from jax import lax
from jax.experimental import pallas as pl
from jax.experimental.pallas import tpu as pltpu
from jax.experimental.pallas import tpu_sc as plsc

NUM_ROWS = 1000000
DIM = 64
BATCH = 4096
SEQ = 200
LANES = 128
NBUF = 4

_INFO = plsc.get_sparse_core_info()
NC = _INFO.num_cores
NS = _INFO.num_subcores
NW = NC * NS
NGRP = SEQ // NBUF


@functools.partial(
    pl.kernel,
    mesh=plsc.VectorSubcoreMesh(core_axis_name="c", subcore_axis_name="s"),
    compiler_params=pltpu.CompilerParams(
        use_tc_tiling_on_sc=True, needs_layout_passes=False
    ),
    out_type=jax.ShapeDtypeStruct((SEQ, DIM, BATCH), jnp.float32),
    scratch_types=[
        pltpu.VMEM((SEQ, LANES), jnp.int32),
    ]
    + [pltpu.VMEM((LANES,), jnp.int32) for _ in range(NBUF)]
    + [pltpu.VMEM((LANES, LANES), jnp.float32) for _ in range(NBUF)]
    + [pltpu.VMEM((DIM, LANES), jnp.float32) for _ in range(NBUF)]
    + [
        pltpu.SemaphoreType.DMA,
        pltpu.SemaphoreType.DMA,
    ],
)
def _emb_lookup(idx_hbm, table_hbm, out_hbm, idx_v, *rest):
    pidx = rest[0:NBUF]
    rows = rest[NBUF:2 * NBUF]
    obuf = rest[2 * NBUF:3 * NBUF]
    gsem, ssem = rest[3 * NBUF], rest[3 * NBUF + 1]

    wid = lax.axis_index("s") * NC + lax.axis_index("c")
    obase = wid * LANES

    pltpu.sync_copy(idx_hbm.at[:, pl.ds(obase, LANES)], idx_v)

    lane16 = lax.iota(jnp.int32, 16)

    def compute_pidx(k, t):
        for lg in range(LANES // 16):
            v = idx_v[t, pl.ds(lg * 16, 16)]
            pidx[k][pl.ds(lg * 16, 16)] = v >> 1

    def transpose_extract(k, t):
        for lg in range(LANES // 16):
            v = idx_v[t, pl.ds(lg * 16, 16)]
            half16 = (v & 1) << 6
            row_idx = lane16 + (lg * 16)

            @plsc.parallel_loop(0, DIM, unroll=8)
            def _(c):
                d_vec = (lane16 + c) & (DIM - 1)
                vals = plsc.load_gather(rows[k], [row_idx, half16 + d_vec])
                plsc.store_scatter(obuf[k], [d_vec, row_idx], vals)

    def body(g, _):
        t0 = g * NBUF
        handles = []
        for k in range(NBUF):
            compute_pidx(k, t0 + k)
            handles.append(
                pltpu.async_copy(table_hbm.at[pidx[k]], rows[k], gsem)
            )
        for k in range(NBUF):
            @pl.when(g > 0)
            def _():
                pltpu.make_async_copy(
                    obuf[k],
                    out_hbm.at[t0 + k - NBUF, :, pl.ds(obase, LANES)],
                    ssem,
                ).wait()

            handles[k].wait()
            transpose_extract(k, t0 + k)
            pltpu.async_copy(
                obuf[k], out_hbm.at[t0 + k, :, pl.ds(obase, LANES)], ssem
            )
        return 0

    lax.fori_loop(0, NGRP, body, 0)
    for k in range(NBUF):
        pltpu.make_async_copy(
            obuf[k], out_hbm.at[SEQ - NBUF + k, :, pl.ds(obase, LANES)], ssem
        ).wait()


def kernel(token_ids, weight):
    idx = token_ids.astype(jnp.int32).T
    table2 = weight.reshape(NUM_ROWS // 2, 2 * DIM)
    out = _emb_lookup(idx, table2)
    return out.transpose(2, 0, 1)


# speedup vs baseline: 2.4617x; 1.0018x over previous
import functools

import jax
import jax.numpy as jnp
from jax import lax
from jax.experimental import pallas as pl
from jax.experimental.pallas import tpu as pltpu
from jax.experimental.pallas import tpu_sc as plsc

NUM_ROWS = 1000000
DIM = 64
BATCH = 4096
SEQ = 200
LANES = 128
NBUF = 4

_INFO = plsc.get_sparse_core_info()
NC = _INFO.num_cores
NS = _INFO.num_subcores
NW = NC * NS
NGRP = SEQ // NBUF


@functools.partial(
    pl.kernel,
    mesh=plsc.VectorSubcoreMesh(core_axis_name="c", subcore_axis_name="s"),
    compiler_params=pltpu.CompilerParams(
        use_tc_tiling_on_sc=True,
        needs_layout_passes=False,
        skip_device_barrier=True,
    ),
    out_type=jax.ShapeDtypeStruct((SEQ, DIM, BATCH), jnp.float32),
    scratch_types=[
        pltpu.VMEM((SEQ, LANES), jnp.int32),
    ]
    + [pltpu.VMEM((LANES,), jnp.int32) for _ in range(NBUF)]
    + [pltpu.VMEM((LANES, LANES), jnp.float32) for _ in range(NBUF)]
    + [pltpu.VMEM((DIM, LANES), jnp.float32) for _ in range(NBUF)]
    + [
        pltpu.SemaphoreType.DMA,
        pltpu.SemaphoreType.DMA,
    ],
)
def _emb_lookup(idx_hbm, table_hbm, out_hbm, idx_v, *rest):
    pidx = rest[0:NBUF]
    rows = rest[NBUF:2 * NBUF]
    obuf = rest[2 * NBUF:3 * NBUF]
    gsem, ssem = rest[3 * NBUF], rest[3 * NBUF + 1]

    wid = lax.axis_index("s") * NC + lax.axis_index("c")
    obase = wid * LANES

    pltpu.sync_copy(idx_hbm.at[:, pl.ds(obase, LANES)], idx_v)

    lane16 = lax.iota(jnp.int32, 16)

    def compute_pidx(k, t):
        for lg in range(LANES // 16):
            v = idx_v[t, pl.ds(lg * 16, 16)]
            pidx[k][pl.ds(lg * 16, 16)] = v >> 1

    def transpose_extract(k, t):
        for lg in range(LANES // 16):
            v = idx_v[t, pl.ds(lg * 16, 16)]
            half16 = (v & 1) << 6
            row_idx = lane16 + (lg * 16)

            @plsc.parallel_loop(0, DIM, unroll=8)
            def _(c):
                d_vec = (lane16 + c) & (DIM - 1)
                vals = plsc.load_gather(rows[k], [row_idx, half16 + d_vec])
                plsc.store_scatter(obuf[k], [d_vec, row_idx], vals)

    def body(g, _):
        t0 = g * NBUF
        handles = []
        for k in range(NBUF):
            compute_pidx(k, t0 + k)
            handles.append(
                pltpu.async_copy(table_hbm.at[pidx[k]], rows[k], gsem)
            )
        for k in range(NBUF):
            @pl.when(g > 0)
            def _():
                pltpu.make_async_copy(
                    obuf[k],
                    out_hbm.at[t0 + k - NBUF, :, pl.ds(obase, LANES)],
                    ssem,
                ).wait()

            handles[k].wait()
            transpose_extract(k, t0 + k)
            pltpu.async_copy(
                obuf[k], out_hbm.at[t0 + k, :, pl.ds(obase, LANES)], ssem
            )
        return 0

    lax.fori_loop(0, NGRP, body, 0)
    for k in range(NBUF):
        pltpu.make_async_copy(
            obuf[k], out_hbm.at[SEQ - NBUF + k, :, pl.ds(obase, LANES)], ssem
        ).wait()


def kernel(token_ids, weight):
    idx = token_ids.astype(jnp.int32).T
    table2 = weight.reshape(NUM_ROWS // 2, 2 * DIM)
    out = _emb_lookup(idx, table2)
    return out.transpose(2, 0, 1)
